# baseline (speedup 1.0000x reference)
import functools

import jax
import jax.numpy as jnp
from jax import lax
from jax.experimental import pallas as pl
from jax.experimental.pallas import tpu as pltpu
from jax.experimental.pallas import tpu_sc as plsc

EMBED_DIM = 1024
BATCH = 4
SEQ = 8192

NC = 2
NS = 16
NW = NC * NS

B_TOTAL = BATCH * SEQ
PER_W = B_TOTAL // NW
CHUNK = 8
NCHUNK = PER_W // CHUNK
NBUF = 8
AHEAD = 5
NG = NCHUNK // NBUF

_mesh = plsc.VectorSubcoreMesh(core_axis_name="c", subcore_axis_name="s")


@functools.partial(
    pl.kernel,
    out_type=jax.ShapeDtypeStruct((B_TOTAL, EMBED_DIM), jnp.float32),
    mesh=_mesh,
    scratch_types=[
        pltpu.VMEM((PER_W,), jnp.int32),
        [pltpu.VMEM((CHUNK, EMBED_DIM), jnp.float32) for _ in range(NBUF)],
        [pltpu.SemaphoreType.DMA for _ in range(NBUF)],
        [pltpu.SemaphoreType.DMA for _ in range(NBUF)],
    ],
)
def _gather_kernel(table_hbm, idx_hbm, out_hbm, idx_v, rows, gsem, wsem):
    wid = lax.axis_index("s") * NC + lax.axis_index("c")
    base = wid * PER_W
    pltpu.sync_copy(
        idx_hbm.at[wid // (SEQ // PER_W),
                   pl.ds((wid % (SEQ // PER_W)) * PER_W, PER_W)], idx_v)

    def start_gather(j, b):
        pltpu.async_copy(table_hbm.at[idx_v.at[pl.ds(j * CHUNK, CHUNK)]],
                         rows[b], gsem[b])

    def wait_gather(b):
        pltpu.make_async_copy(table_hbm.at[pl.ds(0, CHUNK)], rows[b],
                              gsem[b]).wait()

    def start_write(j, b):
        pltpu.async_copy(rows[b], out_hbm.at[pl.ds(base + j * CHUNK, CHUNK)],
                         wsem[b])

    def wait_write(b):
        pltpu.make_async_copy(rows[b], out_hbm.at[pl.ds(base, CHUNK)],
                              wsem[b]).wait()

    for k in range(AHEAD):
        start_gather(k, k)

    def group(g, carry):
        for k in range(NBUF):
            j = g * NBUF + k
            b = k
            bn = (k + AHEAD) % NBUF
            wait_gather(b)
            start_write(j, b)
            jn = j + AHEAD
            if k < NBUF - AHEAD:
                @pl.when(j >= NBUF - AHEAD)
                def _():
                    wait_write(bn)
            else:
                wait_write(bn)

            @pl.when(jn < NCHUNK)
            def _():
                start_gather(jn, bn)
        return carry

    lax.fori_loop(0, NG, group, 0)
    for j in range(NCHUNK - (NBUF - AHEAD), NCHUNK):
        wait_write(j % NBUF)


def kernel(input_pos_tensors, table):
    out = _gather_kernel(table, input_pos_tensors.astype(jnp.int32))
    return jnp.reshape(out, (BATCH, SEQ, EMBED_DIM))


# speedup vs baseline: 1.0001x; 1.0001x over previous
import functools

import jax
import jax.numpy as jnp
from jax import lax
from jax.experimental import pallas as pl
from jax.experimental.pallas import tpu as pltpu
from jax.experimental.pallas import tpu_sc as plsc

EMBED_DIM = 1024
BATCH = 4
SEQ = 8192

NC = 2
NS = 16
NW = NC * NS

B_TOTAL = BATCH * SEQ
PER_W = B_TOTAL // NW
CHUNK = 8
NCHUNK = PER_W // CHUNK
NBUF = 8
WG = 4
WSLOT = 2
NG = NCHUNK // NBUF

_mesh = plsc.VectorSubcoreMesh(core_axis_name="c", subcore_axis_name="s")


@functools.partial(
    pl.kernel,
    out_type=jax.ShapeDtypeStruct((B_TOTAL, EMBED_DIM), jnp.float32),
    mesh=_mesh,
    scratch_types=[
        pltpu.VMEM((PER_W,), jnp.int32),
        pltpu.VMEM((NBUF * CHUNK, EMBED_DIM), jnp.float32),
        [pltpu.SemaphoreType.DMA for _ in range(NBUF)],
        [pltpu.SemaphoreType.DMA for _ in range(WSLOT)],
    ],
)
def _gather_kernel(table_hbm, idx_hbm, out_hbm, idx_v, rows, gsem, wsem):
    wid = lax.axis_index("s") * NC + lax.axis_index("c")
    base = wid * PER_W
    pltpu.sync_copy(
        idx_hbm.at[wid // (SEQ // PER_W),
                   pl.ds((wid % (SEQ // PER_W)) * PER_W, PER_W)], idx_v)

    def start_gather(j, b):
        pltpu.async_copy(table_hbm.at[idx_v.at[pl.ds(j * CHUNK, CHUNK)]],
                         rows.at[pl.ds(b * CHUNK, CHUNK)], gsem[b])

    def wait_gather(b):
        pltpu.make_async_copy(table_hbm.at[pl.ds(0, CHUNK)],
                              rows.at[pl.ds(b * CHUNK, CHUNK)], gsem[b]).wait()

    def start_write(m, s):
        pltpu.async_copy(rows.at[pl.ds(s * WG * CHUNK, WG * CHUNK)],
                         out_hbm.at[pl.ds(base + m * WG * CHUNK, WG * CHUNK)],
                         wsem[s])

    def wait_write(s):
        pltpu.make_async_copy(rows.at[pl.ds(s * WG * CHUNK, WG * CHUNK)],
                              out_hbm.at[pl.ds(base, WG * CHUNK)],
                              wsem[s]).wait()

    for k in range(WG):
        start_gather(k, k)

    def group(g, carry):
        for k in range(NBUF):
            j = g * NBUF + k
            wait_gather(k)
            if k % WG == WG - 1:
                start_write(j // WG, (k // WG) % WSLOT)
            jn = j + WG
            if k % WG == 0:
                @pl.when(jn >= 2 * WG)
                def _():
                    wait_write((k // WG + 1) % WSLOT)

            @pl.when(jn < NCHUNK)
            def _():
                start_gather(jn, (k + WG) % NBUF)
        return carry

    lax.fori_loop(0, NG, group, 0)
    wait_write((NCHUNK // WG - 1) % WSLOT)


def kernel(input_pos_tensors, table):
    out = _gather_kernel(table, input_pos_tensors.astype(jnp.int32))
    return jnp.reshape(out, (BATCH, SEQ, EMBED_DIM))


# speedup vs baseline: 1.0030x; 1.0029x over previous
import functools

import jax
import jax.numpy as jnp
from jax import lax
from jax.experimental import pallas as pl
from jax.experimental.pallas import tpu as pltpu
from jax.experimental.pallas import tpu_sc as plsc

EMBED_DIM = 1024
BATCH = 4
SEQ = 8192

NC = 2
NS = 16
NW = NC * NS

B_TOTAL = BATCH * SEQ
PER_W = B_TOTAL // NW
CHUNK = 8
NCHUNK = PER_W // CHUNK
NBUF = 8
AHEAD = 5
NG = NCHUNK // NBUF

_mesh = plsc.VectorSubcoreMesh(core_axis_name="c", subcore_axis_name="s")


@functools.partial(
    pl.kernel,
    out_type=jax.ShapeDtypeStruct((B_TOTAL, EMBED_DIM), jnp.float32),
    mesh=_mesh,
    scratch_types=[
        pltpu.VMEM((PER_W,), jnp.int32),
        [pltpu.VMEM((CHUNK, EMBED_DIM), jnp.float32) for _ in range(NBUF)],
        [pltpu.SemaphoreType.DMA for _ in range(NBUF)],
        [pltpu.SemaphoreType.DMA for _ in range(NBUF)],
    ],
)
def _gather_kernel(table_hbm, idx_hbm, out_hbm, idx_v, rows, gsem, wsem):
    wid = lax.axis_index("s") * NC + lax.axis_index("c")
    base = wid * PER_W
    pltpu.sync_copy(
        idx_hbm.at[wid // (SEQ // PER_W),
                   pl.ds((wid % (SEQ // PER_W)) * PER_W, PER_W)], idx_v)

    def start_gather(j, b):
        pltpu.async_copy(table_hbm.at[idx_v.at[pl.ds(j * CHUNK, CHUNK)]],
                         rows[b], gsem[b])

    def wait_gather(b):
        pltpu.make_async_copy(table_hbm.at[pl.ds(0, CHUNK)], rows[b],
                              gsem[b]).wait()

    def start_write(j, b):
        pltpu.async_copy(rows[b], out_hbm.at[pl.ds(base + j * CHUNK, CHUNK)],
                         wsem[b])

    def wait_write(b):
        pltpu.make_async_copy(rows[b], out_hbm.at[pl.ds(base, CHUNK)],
                              wsem[b]).wait()

    for k in range(AHEAD):
        start_gather(k, k)

    def group(g, carry):
        for k in range(NBUF):
            j = g * NBUF + k
            b = k
            bn = (k + AHEAD) % NBUF
            wait_gather(b)
            start_write(j, b)
            jn = j + AHEAD
            if k < NBUF - AHEAD:
                @pl.when(j >= NBUF - AHEAD)
                def _():
                    wait_write(bn)
            else:
                wait_write(bn)

            @pl.when(jn < NCHUNK)
            def _():
                start_gather(jn, bn)
        return carry

    lax.fori_loop(0, NG, group, 0)
    for j in range(NCHUNK - (NBUF - AHEAD), NCHUNK):
        wait_write(j % NBUF)


def kernel(input_pos_tensors, table):
    out = _gather_kernel(table, input_pos_tensors.astype(jnp.int32))
    return jnp.reshape(out, (BATCH, SEQ, EMBED_DIM))
